# baseline (speedup 1.0000x reference)
import functools

import jax
import jax.numpy as jnp
from jax import lax
from jax.experimental import pallas as pl
from jax.experimental.pallas import tpu as pltpu
from jax.experimental.pallas import tpu_sc as plsc

F = 4
V = 100002
D = 64
DP = 128
NC = 2
NS = 16
NW = NC * NS
CH = 80


def _make_sc_kernel(n_tok):
    per_w = n_tok // NW
    nch = per_w // CH
    mesh = plsc.VectorSubcoreMesh(core_axis_name="c", subcore_axis_name="s")

    @functools.partial(
        pl.kernel,
        out_type=[
            jax.ShapeDtypeStruct((n_tok, DP), jnp.float32),
            jax.ShapeDtypeStruct((n_tok,), jnp.int32),
        ],
        mesh=mesh,
        scratch_types=[
            pltpu.VMEM((per_w,), jnp.int32),
            pltpu.VMEM((CH,), jnp.int32),
            pltpu.VMEM((CH, DP), jnp.float32),
            pltpu.SemaphoreType.DMA,
        ],
    )
    def sc_kernel(tok_hbm, map_hbm, tab_hbm, out_hbm, oidx_hbm,
                  tok_v, fidx_v, rows_v, sem):
        c = lax.axis_index("c")
        s = lax.axis_index("s")
        wid = s * NC + c
        tbase = wid * per_w
        pltpu.sync_copy(tok_hbm.at[pl.ds(tbase, per_w)], tok_v)

        def chunk(j, carry):
            cb = j * CH
            pltpu.async_copy(map_hbm.at[tok_v.at[pl.ds(cb, CH)]], fidx_v, sem).wait()
            pltpu.sync_copy(fidx_v, oidx_hbm.at[pl.ds(tbase + cb, CH)])
            pltpu.async_copy(tab_hbm.at[fidx_v], rows_v, sem).wait()
            pltpu.sync_copy(rows_v, out_hbm.at[pl.ds(tbase + cb, CH)])
            return carry

        lax.fori_loop(0, nch, chunk, 0)

    return sc_kernel


@jax.jit
def kernel(token_seqs, tables, mappings):
    b, s = token_seqs.shape
    n_tok = b * s
    tok_flat = token_seqs.reshape(n_tok)
    sck = _make_sc_kernel(n_tok)
    outs = []
    idxs = []
    for f in range(F):
        tab128_f = jnp.pad(tables[f], ((0, 0), (0, DP - D)))
        out_pad_f, oidx_f = sck(tok_flat, mappings[f], tab128_f)
        outs.append(out_pad_f[:, :D].reshape(b, s, D))
        idxs.append(oidx_f.reshape(b, s))
    out_tensor = jnp.stack(outs)
    out_indices = jnp.stack(idxs)
    return (out_tensor, out_indices)
